# baseline (speedup 1.0000x reference)
import functools

import jax
import jax.numpy as jnp
from jax import lax
from jax.experimental import pallas as pl
from jax.experimental.pallas import tpu as pltpu
from jax.experimental.pallas import tpu_sc as plsc

B, H, W = 16, 512, 512
NPOS, NNEG = 256, 256
PTS_PER_B = NPOS + NNEG
P = B * PTS_PER_B
NW = 32
PPW = P // NW
LANES = 16
GCHUNK = 128


def _sc_sample_kernel():
    mesh = plsc.VectorSubcoreMesh(core_axis_name="c", subcore_axis_name="s")

    @functools.partial(
        pl.kernel,
        mesh=mesh,
        out_type=jax.ShapeDtypeStruct((P,), jnp.float32),
        scratch_types=[
            pltpu.VMEM((2 * PPW,), jnp.int32),
            pltpu.VMEM((PPW,), jnp.int32),
            pltpu.VMEM((PPW,), jnp.float32),
            pltpu.SemaphoreType.DMA,
        ],
    )
    def sample(mask_hbm, pos_hbm, neg_hbm, out_hbm, ptv, idxv, sv, sem):
        c = lax.axis_index("c")
        s = lax.axis_index("s")
        wid = s * 2 + c
        base_pt = wid * PPW
        batch = wid // 2
        half = wid % 2
        mask_base = batch * (H * W)

        coord_base = batch * (2 * PPW)

        @pl.when(half == 0)
        def _():
            pltpu.sync_copy(pos_hbm.at[pl.ds(coord_base, 2 * PPW)], ptv)

        @pl.when(half == 1)
        def _():
            pltpu.sync_copy(neg_hbm.at[pl.ds(coord_base, 2 * PPW)], ptv)

        lanes = lax.iota(jnp.int32, LANES)
        idx_e = (lanes & 7) << 1
        idx_o = idx_e + 1
        lo_half = lanes < 8
        dnums = lax.GatherDimensionNumbers(
            offset_dims=(), collapsed_slice_dims=(0,), start_index_map=(0,))

        def _permute(v, idx):
            return lax.gather(v, idx[:, None], dnums, slice_sizes=(1,),
                              mode=lax.GatherScatterMode.PROMISE_IN_BOUNDS)

        for j in range(PPW // LANES):
            v_lo = ptv[pl.ds(j * 2 * LANES, LANES)]
            v_hi = ptv[pl.ds(j * 2 * LANES + LANES, LANES)]
            x = jnp.where(lo_half, _permute(v_lo, idx_e), _permute(v_hi, idx_e))
            y = jnp.where(lo_half, _permute(v_lo, idx_o), _permute(v_hi, idx_o))
            idxv[pl.ds(j * LANES, LANES)] = (
                mask_base + ((y >> 3) << 12) + ((y & 7) << 7)
                + ((x >> 7) << 10) + (x & 127))

        copies = []
        for k in range(PPW // GCHUNK):
            gsl = pl.ds(k * GCHUNK, GCHUNK)
            copies.append(
                pltpu.async_copy(mask_hbm.at[idxv.at[gsl]], sv.at[gsl], sem))
        for cp in copies:
            cp.wait()

        pltpu.sync_copy(sv, out_hbm.at[pl.ds(base_pt, PPW)])

    return sample


_sc_sample = _sc_sample_kernel()

_ROWS, _COLS = P // 128, 128


def _bce_mean_body(s_ref, o_ref):
    s = s_ref[...]
    row = lax.broadcasted_iota(jnp.int32, (_ROWS, _COLS), 0)
    tgt = jnp.where((row & 3) < 2, 1.0, 0.0)
    bce = jnp.maximum(s, 0.0) - s * tgt + jnp.log1p(jnp.exp(-jnp.abs(s)))
    o_ref[...] = (jnp.sum(bce) * (1.0 / float(P))).reshape(1, 1)


def kernel(pred_mask, positive_points, negative_points):
    mask_flat = (
        pred_mask.reshape(B, 1, H // 8, 8, W // 128, 128)
        .transpose(0, 1, 2, 4, 3, 5)
        .reshape(-1)
    )
    pos_flat = positive_points.astype(jnp.int32).reshape(-1)
    neg_flat = negative_points.astype(jnp.int32).reshape(-1)

    samples = _sc_sample(mask_flat, pos_flat, neg_flat)

    loss = pl.pallas_call(
        _bce_mean_body,
        out_shape=jax.ShapeDtypeStruct((1, 1), jnp.float32),
    )(samples.reshape(_ROWS, _COLS))
    return loss[0, 0]


# speedup vs baseline: 1.1221x; 1.1221x over previous
import functools

import jax
import jax.numpy as jnp
from jax import lax
from jax.experimental import pallas as pl
from jax.experimental.pallas import tpu as pltpu
from jax.experimental.pallas import tpu_sc as plsc

B, H, W = 16, 512, 512
NPOS, NNEG = 256, 256
PTS_PER_B = NPOS + NNEG
P = B * PTS_PER_B
NW = 32
PPW = P // NW
LANES = 16
GCHUNK = 128


def _sc_sample_kernel():
    mesh = plsc.VectorSubcoreMesh(core_axis_name="c", subcore_axis_name="s")

    @functools.partial(
        pl.kernel,
        mesh=mesh,
        out_type=jax.ShapeDtypeStruct((P,), jnp.float32),
        scratch_types=[
            pltpu.VMEM((PPW,), jnp.int32),
            pltpu.VMEM((PPW,), jnp.int32),
            pltpu.VMEM((PPW,), jnp.float32),
            pltpu.SemaphoreType.DMA,
        ],
    )
    def sample(mask_hbm, pos_hbm, neg_hbm, out_hbm, ptv, idxv, sv, sem):
        c = lax.axis_index("c")
        s = lax.axis_index("s")
        wid = s * 2 + c
        base_pt = wid * PPW
        batch = wid // 2
        half = wid % 2
        mask_base = batch * (H * W)

        coord_base = batch * PPW

        @pl.when(half == 0)
        def _():
            pltpu.sync_copy(pos_hbm.at[pl.ds(coord_base, PPW)], ptv)

        @pl.when(half == 1)
        def _():
            pltpu.sync_copy(neg_hbm.at[pl.ds(coord_base, PPW)], ptv)

        for j in range(PPW // LANES):
            v = ptv[pl.ds(j * LANES, LANES)]
            x = v & 0xFFFF
            y = v >> 16
            idxv[pl.ds(j * LANES, LANES)] = (
                mask_base + ((y >> 3) << 12) + ((y & 7) << 7)
                + ((x >> 7) << 10) + (x & 127))

        copies = []
        for k in range(PPW // GCHUNK):
            gsl = pl.ds(k * GCHUNK, GCHUNK)
            copies.append(
                pltpu.async_copy(mask_hbm.at[idxv.at[gsl]], sv.at[gsl], sem))
        for cp in copies:
            cp.wait()

        pltpu.sync_copy(sv, out_hbm.at[pl.ds(base_pt, PPW)])

    return sample


_sc_sample = _sc_sample_kernel()

_ROWS, _COLS = P // 128, 128


def _bce_mean_body(s_ref, o_ref):
    s = s_ref[...]
    row = lax.broadcasted_iota(jnp.int32, (_ROWS, _COLS), 0)
    tgt = jnp.where((row & 3) < 2, 1.0, 0.0)
    bce = jnp.maximum(s, 0.0) - s * tgt + jnp.log1p(jnp.exp(-jnp.abs(s)))
    o_ref[...] = (jnp.sum(bce) * (1.0 / float(P))).reshape(1, 1)


def kernel(pred_mask, positive_points, negative_points):
    mask_flat = (
        pred_mask.reshape(B, 1, H // 8, 8, W // 128, 128)
        .transpose(0, 1, 2, 4, 3, 5)
        .reshape(-1)
    )
    pp = positive_points.astype(jnp.int32)
    np_ = negative_points.astype(jnp.int32)
    pos_flat = (pp[:, :, 0] | (pp[:, :, 1] << 16)).reshape(-1)
    neg_flat = (np_[:, :, 0] | (np_[:, :, 1] << 16)).reshape(-1)

    samples = _sc_sample(mask_flat, pos_flat, neg_flat)

    loss = pl.pallas_call(
        _bce_mean_body,
        out_shape=jax.ShapeDtypeStruct((1, 1), jnp.float32),
    )(samples.reshape(_ROWS, _COLS))
    return loss[0, 0]
